# baseline (speedup 1.0000x reference)
import jax
import jax.numpy as jnp
from jax.experimental import pallas as pl
from jax.experimental.pallas import tpu as pltpu


def _fused_body(x_ref, d_ref, w1_ref, w2_ref, k_ref,
                o_ref, b_ref, raw_ref, m_ref, s_ref):
    x = x_ref[...]
    s = jnp.sum(x * d_ref[...][None], axis=-1)
    s_ref[...] = s

    h = jnp.maximum(jnp.dot(s, w1_ref[...].T, preferred_element_type=jnp.float32), 0.0)
    raw = jnp.dot(h, w2_ref[...].T, preferred_element_type=jnp.float32)
    raw_ref[...] = raw
    bounded = jnp.tanh(raw)
    b_ref[...] = bounded

    c = raw.shape[1]
    bits = jax.lax.bitcast_convert_type(raw + 0.0, jnp.int32)
    key = bits ^ ((bits >> 31) & jnp.int32(0x7FFFFFFF))
    k = k_ref[...]

    cnt0 = jnp.sum((key >= 0).astype(jnp.int32), axis=1, keepdims=True)
    base = jnp.where(cnt0 >= k, jnp.int32(0), jnp.int32(-2147483648))
    for bit in range(30, -1, -1):
        cand = base + jnp.int32(1 << bit)
        cnt = jnp.sum((key >= cand).astype(jnp.int32), axis=1, keepdims=True)
        base = jnp.where(cnt >= k, cand, base)

    gt = key > base
    eq = key == base
    cnt_gt = jnp.sum(gt.astype(jnp.int32), axis=1, keepdims=True)
    rows = jax.lax.broadcasted_iota(jnp.int32, (c, c), 0)
    cols = jax.lax.broadcasted_iota(jnp.int32, (c, c), 1)
    tri = (rows < cols).astype(jnp.float32)
    eq_prefix = jnp.dot(eq.astype(jnp.float32), tri,
                        preferred_element_type=jnp.float32)
    n_eq = (k - cnt_gt).astype(jnp.float32)
    mask = jnp.where(gt | (eq & (eq_prefix < n_eq)), 1.0, 0.0)
    m_ref[...] = mask

    gate = bounded + (mask - bounded)
    o_ref[...] = x * gate[:, :, None]


def kernel(x, dct_weight, W1, W2, k_tensor):
    b, c, dh, dw = x.shape
    p = dh * dw
    hid = W1.shape[0]
    x3 = x.reshape(b, c, p)
    dct2 = dct_weight.reshape(c, p)
    k2d = k_tensor.reshape(b, 1)

    bb = 8
    grid = (b // bb,)
    f = jnp.float32

    out3, bounded, raw, mask, s = pl.pallas_call(
        _fused_body,
        grid=grid,
        in_specs=[
            pl.BlockSpec((bb, c, p), lambda i: (i, 0, 0)),
            pl.BlockSpec((c, p), lambda i: (0, 0)),
            pl.BlockSpec((hid, c), lambda i: (0, 0)),
            pl.BlockSpec((c, hid), lambda i: (0, 0)),
            pl.BlockSpec((bb, 1), lambda i: (i, 0)),
        ],
        out_specs=(
            pl.BlockSpec((bb, c, p), lambda i: (i, 0, 0)),
            pl.BlockSpec((bb, c), lambda i: (i, 0)),
            pl.BlockSpec((bb, c), lambda i: (i, 0)),
            pl.BlockSpec((bb, c), lambda i: (i, 0)),
            pl.BlockSpec((bb, c), lambda i: (i, 0)),
        ),
        out_shape=(
            jax.ShapeDtypeStruct((b, c, p), f),
            jax.ShapeDtypeStruct((b, c), f),
            jax.ShapeDtypeStruct((b, c), f),
            jax.ShapeDtypeStruct((b, c), f),
            jax.ShapeDtypeStruct((b, c), f),
        ),
        compiler_params=pltpu.CompilerParams(
            dimension_semantics=("parallel",),
            vmem_limit_bytes=120 * 1024 * 1024),
    )(x3, dct2, W1, W2, k2d)

    return (out3.reshape(b, c, dh, dw), bounded, raw, mask, s)


# speedup vs baseline: 1.1652x; 1.1652x over previous
import jax
import jax.numpy as jnp
from jax.experimental import pallas as pl
from jax.experimental.pallas import tpu as pltpu


def _fused_body(x_ref, d_ref, w1_ref, w2_ref, k_ref,
                o_ref, b_ref, raw_ref, m_ref, s_ref):
    x = x_ref[...]
    s = jnp.sum(x * d_ref[...][None], axis=-1)
    s_ref[...] = s

    h = jnp.maximum(jnp.dot(s, w1_ref[...].T, preferred_element_type=jnp.float32), 0.0)
    raw = jnp.dot(h, w2_ref[...].T, preferred_element_type=jnp.float32)
    raw_ref[...] = raw
    bounded = jnp.tanh(raw)
    b_ref[...] = bounded

    c = raw.shape[1]
    bits = jax.lax.bitcast_convert_type(raw + 0.0, jnp.int32)
    key = bits ^ ((bits >> 31) & jnp.int32(0x7FFFFFFF))
    k = k_ref[...]

    cnt0 = jnp.sum((key >= 0).astype(jnp.int32), axis=1, keepdims=True)
    base = jnp.where(cnt0 >= k, jnp.int32(0), jnp.int32(-2147483648))
    for bit in range(30, -1, -1):
        cand = base + jnp.int32(1 << bit)
        cnt = jnp.sum((key >= cand).astype(jnp.int32), axis=1, keepdims=True)
        base = jnp.where(cnt >= k, cand, base)

    gt = key > base
    eq = key == base
    cnt_gt = jnp.sum(gt.astype(jnp.int32), axis=1, keepdims=True)
    rows = jax.lax.broadcasted_iota(jnp.int32, (c, c), 0)
    cols = jax.lax.broadcasted_iota(jnp.int32, (c, c), 1)
    tri = (rows < cols).astype(jnp.float32)
    eq_prefix = jnp.dot(eq.astype(jnp.float32), tri,
                        preferred_element_type=jnp.float32)
    n_eq = (k - cnt_gt).astype(jnp.float32)
    mask = jnp.where(gt | (eq & (eq_prefix < n_eq)), 1.0, 0.0)
    m_ref[...] = mask

    gate = bounded + (mask - bounded)
    o_ref[...] = x * gate[:, :, None]


def kernel(x, dct_weight, W1, W2, k_tensor):
    b, c, dh, dw = x.shape
    p = dh * dw
    hid = W1.shape[0]
    x3 = x.reshape(b, c, p)
    dct2 = dct_weight.reshape(c, p)
    k2d = k_tensor.reshape(b, 1)

    bb = 16
    grid = (b // bb,)
    f = jnp.float32

    out3, bounded, raw, mask, s = pl.pallas_call(
        _fused_body,
        grid=grid,
        in_specs=[
            pl.BlockSpec((bb, c, p), lambda i: (i, 0, 0)),
            pl.BlockSpec((c, p), lambda i: (0, 0)),
            pl.BlockSpec((hid, c), lambda i: (0, 0)),
            pl.BlockSpec((c, hid), lambda i: (0, 0)),
            pl.BlockSpec((bb, 1), lambda i: (i, 0)),
        ],
        out_specs=(
            pl.BlockSpec((bb, c, p), lambda i: (i, 0, 0)),
            pl.BlockSpec((bb, c), lambda i: (i, 0)),
            pl.BlockSpec((bb, c), lambda i: (i, 0)),
            pl.BlockSpec((bb, c), lambda i: (i, 0)),
            pl.BlockSpec((bb, c), lambda i: (i, 0)),
        ),
        out_shape=(
            jax.ShapeDtypeStruct((b, c, p), f),
            jax.ShapeDtypeStruct((b, c), f),
            jax.ShapeDtypeStruct((b, c), f),
            jax.ShapeDtypeStruct((b, c), f),
            jax.ShapeDtypeStruct((b, c), f),
        ),
        compiler_params=pltpu.CompilerParams(
            dimension_semantics=("parallel",),
            vmem_limit_bytes=120 * 1024 * 1024),
    )(x3, dct2, W1, W2, k2d)

    return (out3.reshape(b, c, dh, dw), bounded, raw, mask, s)


# speedup vs baseline: 1.2190x; 1.0462x over previous
import jax
import jax.numpy as jnp
from jax.experimental import pallas as pl
from jax.experimental.pallas import tpu as pltpu


def _topk_mask_general(raw, k):
    c = raw.shape[1]
    bits = jax.lax.bitcast_convert_type(raw + 0.0, jnp.int32)
    key = bits ^ ((bits >> 31) & jnp.int32(0x7FFFFFFF))

    cnt0 = jnp.sum((key >= 0).astype(jnp.int32), axis=1, keepdims=True)
    base = jnp.where(cnt0 >= k, jnp.int32(0), jnp.int32(-2147483648))
    for bit in range(30, -1, -1):
        cand = base + jnp.int32(1 << bit)
        cnt = jnp.sum((key >= cand).astype(jnp.int32), axis=1, keepdims=True)
        base = jnp.where(cnt >= k, cand, base)

    gt = key > base
    eq = key == base
    cnt_gt = jnp.sum(gt.astype(jnp.int32), axis=1, keepdims=True)
    rows = jax.lax.broadcasted_iota(jnp.int32, (c, c), 0)
    cols = jax.lax.broadcasted_iota(jnp.int32, (c, c), 1)
    tri = (rows < cols).astype(jnp.float32)
    eq_prefix = jnp.dot(eq.astype(jnp.float32), tri,
                        preferred_element_type=jnp.float32)
    n_eq = (k - cnt_gt).astype(jnp.float32)
    return jnp.where(gt | (eq & (eq_prefix < n_eq)), 1.0, 0.0)


def _fused_body(x_ref, d_ref, w1_ref, w2_ref, k_ref,
                o_ref, b_ref, raw_ref, m_ref, s_ref, m_scr):
    x = x_ref[...]
    s = jnp.sum(x * d_ref[...][None], axis=-1)
    s_ref[...] = s

    h = jnp.maximum(jnp.dot(s, w1_ref[...].T, preferred_element_type=jnp.float32), 0.0)
    raw = jnp.dot(h, w2_ref[...].T, preferred_element_type=jnp.float32)
    raw_ref[...] = raw
    bounded = jnp.tanh(raw)
    b_ref[...] = bounded

    bb, c = raw.shape
    k = k_ref[...]

    rmax = jnp.max(raw, axis=1, keepdims=True)
    rmin = jnp.min(raw, axis=1, keepdims=True)
    n_varying = jnp.sum((rmax != rmin).astype(jnp.int32))

    @pl.when(n_varying == 0)
    def _fast():
        cols = jax.lax.broadcasted_iota(jnp.int32, (bb, c), 1)
        m_scr[...] = (cols < k).astype(jnp.float32)

    @pl.when(n_varying != 0)
    def _general():
        m_scr[...] = _topk_mask_general(raw, k)

    mask = m_scr[...]
    m_ref[...] = mask
    gate = bounded + (mask - bounded)
    o_ref[...] = x * gate[:, :, None]


def kernel(x, dct_weight, W1, W2, k_tensor):
    b, c, dh, dw = x.shape
    p = dh * dw
    hid = W1.shape[0]
    x3 = x.reshape(b, c, p)
    dct2 = dct_weight.reshape(c, p)
    k2d = k_tensor.reshape(b, 1)

    bb = 16
    grid = (b // bb,)
    f = jnp.float32

    out3, bounded, raw, mask, s = pl.pallas_call(
        _fused_body,
        grid=grid,
        in_specs=[
            pl.BlockSpec((bb, c, p), lambda i: (i, 0, 0)),
            pl.BlockSpec((c, p), lambda i: (0, 0)),
            pl.BlockSpec((hid, c), lambda i: (0, 0)),
            pl.BlockSpec((c, hid), lambda i: (0, 0)),
            pl.BlockSpec((bb, 1), lambda i: (i, 0)),
        ],
        out_specs=(
            pl.BlockSpec((bb, c, p), lambda i: (i, 0, 0)),
            pl.BlockSpec((bb, c), lambda i: (i, 0)),
            pl.BlockSpec((bb, c), lambda i: (i, 0)),
            pl.BlockSpec((bb, c), lambda i: (i, 0)),
            pl.BlockSpec((bb, c), lambda i: (i, 0)),
        ),
        out_shape=(
            jax.ShapeDtypeStruct((b, c, p), f),
            jax.ShapeDtypeStruct((b, c), f),
            jax.ShapeDtypeStruct((b, c), f),
            jax.ShapeDtypeStruct((b, c), f),
            jax.ShapeDtypeStruct((b, c), f),
        ),
        scratch_shapes=[pltpu.VMEM((bb, c), jnp.float32)],
        compiler_params=pltpu.CompilerParams(
            dimension_semantics=("parallel",),
            vmem_limit_bytes=120 * 1024 * 1024),
    )(x3, dct2, W1, W2, k2d)

    return (out3.reshape(b, c, dh, dw), bounded, raw, mask, s)


# speedup vs baseline: 1.2285x; 1.0078x over previous
import jax
import jax.numpy as jnp
from jax.experimental import pallas as pl
from jax.experimental.pallas import tpu as pltpu


def _topk_mask_general(raw, k):
    c = raw.shape[1]
    bits = jax.lax.bitcast_convert_type(raw + 0.0, jnp.int32)
    key = bits ^ ((bits >> 31) & jnp.int32(0x7FFFFFFF))

    cnt0 = jnp.sum((key >= 0).astype(jnp.int32), axis=1, keepdims=True)
    base = jnp.where(cnt0 >= k, jnp.int32(0), jnp.int32(-2147483648))
    for bit in range(30, -1, -1):
        cand = base + jnp.int32(1 << bit)
        cnt = jnp.sum((key >= cand).astype(jnp.int32), axis=1, keepdims=True)
        base = jnp.where(cnt >= k, cand, base)

    gt = key > base
    eq = key == base
    cnt_gt = jnp.sum(gt.astype(jnp.int32), axis=1, keepdims=True)
    rows = jax.lax.broadcasted_iota(jnp.int32, (c, c), 0)
    cols = jax.lax.broadcasted_iota(jnp.int32, (c, c), 1)
    tri = (rows < cols).astype(jnp.float32)
    eq_prefix = jnp.dot(eq.astype(jnp.float32), tri,
                        preferred_element_type=jnp.float32)
    n_eq = (k - cnt_gt).astype(jnp.float32)
    return jnp.where(gt | (eq & (eq_prefix < n_eq)), 1.0, 0.0)


def _fused_body(x_ref, d_ref, w1_ref, w2_ref, k_ref,
                o_ref, b_ref, raw_ref, m_ref, s_ref, m_scr):
    bb = x_ref.shape[0]
    rs = pl.ds(pl.program_id(0) * bb, bb)

    s = jnp.sum(x_ref[...] * d_ref[...][None], axis=-1)
    s_ref[rs, :] = s

    h = jnp.maximum(jnp.dot(s, w1_ref[...].T, preferred_element_type=jnp.float32), 0.0)
    raw = jnp.dot(h, w2_ref[...].T, preferred_element_type=jnp.float32)
    raw_ref[rs, :] = raw
    bounded = jnp.tanh(raw)
    b_ref[rs, :] = bounded

    c = raw.shape[1]
    k = k_ref[rs, :]

    rmax = jnp.max(raw, axis=1, keepdims=True)
    rmin = jnp.min(raw, axis=1, keepdims=True)
    n_varying = jnp.sum((rmax != rmin).astype(jnp.int32))

    @pl.when(n_varying == 0)
    def _fast():
        cols = jax.lax.broadcasted_iota(jnp.int32, (bb, c), 1)
        m_scr[...] = (cols < k).astype(jnp.float32)

    @pl.when(n_varying != 0)
    def _general():
        m_scr[...] = _topk_mask_general(raw, k)

    mask = m_scr[...]
    m_ref[rs, :] = mask
    gate = bounded + (mask - bounded)
    o_ref[...] = x_ref[...] * gate[:, :, None]


def kernel(x, dct_weight, W1, W2, k_tensor):
    b, c, dh, dw = x.shape
    p = dh * dw
    hid = W1.shape[0]
    x3 = x.reshape(b, c, p)
    dct2 = dct_weight.reshape(c, p)
    k2d = k_tensor.reshape(b, 1)

    bb = 16
    grid = (b // bb,)
    f = jnp.float32
    full_bc = pl.BlockSpec((b, c), lambda i: (0, 0))

    out3, bounded, raw, mask, s = pl.pallas_call(
        _fused_body,
        grid=grid,
        in_specs=[
            pl.BlockSpec((bb, c, p), lambda i: (i, 0, 0)),
            pl.BlockSpec((c, p), lambda i: (0, 0)),
            pl.BlockSpec((hid, c), lambda i: (0, 0)),
            pl.BlockSpec((c, hid), lambda i: (0, 0)),
            pl.BlockSpec((b, 1), lambda i: (0, 0)),
        ],
        out_specs=(
            pl.BlockSpec((bb, c, p), lambda i: (i, 0, 0)),
            full_bc, full_bc, full_bc, full_bc,
        ),
        out_shape=(
            jax.ShapeDtypeStruct((b, c, p), f),
            jax.ShapeDtypeStruct((b, c), f),
            jax.ShapeDtypeStruct((b, c), f),
            jax.ShapeDtypeStruct((b, c), f),
            jax.ShapeDtypeStruct((b, c), f),
        ),
        scratch_shapes=[pltpu.VMEM((bb, c), jnp.float32)],
        compiler_params=pltpu.CompilerParams(
            dimension_semantics=("arbitrary",),
            vmem_limit_bytes=120 * 1024 * 1024),
    )(x3, dct2, W1, W2, k2d)

    return (out3.reshape(b, c, dh, dw), bounded, raw, mask, s)
